# Initial kernel scaffold; baseline (speedup 1.0000x reference)
import jax
import jax.numpy as jnp
from jax.experimental import pallas as pl


def kernel(target, context, negative_samples, target_weight, context_weight):
    raise NotImplementedError("write your pallas kernel here")


# speedup vs baseline: 1.8323x; 1.8323x over previous
import jax
import jax.numpy as jnp
from jax import lax
from jax.experimental import pallas as pl
from jax.experimental.pallas import tpu as pltpu
from jax.experimental.pallas import tpu_sc as plsc

V = 1000000
V2 = V // 2
B = 16384
K = 20
D = 64
NC = 2
NS = 16
NW = NC * NS
BPW = B // NW
C = 16
NCHUNK = BPW // C
NPAIR = NCHUNK // 2
UNR = 4
KH = K // 2

NBLK_FULL = V // 128
BPT = NBLK_FULL // NW
NBLK_REM = NBLK_FULL - BPT * NW

_params = pltpu.CompilerParams(
    needs_layout_passes=False, use_tc_tiling_on_sc=True)
_mesh = plsc.VectorSubcoreMesh(
    core_axis_name="c", subcore_axis_name="s", num_cores=NC, num_subcores=NS)


def _sc_transpose(twT, cwT, tail_t, tail_c, tw2, cw2,
                  in_a, in_b, out_a, out_b, sem_ia, sem_ib, sem_oa, sem_ob):
    wid = lax.axis_index("s") * NC + lax.axis_index("c")
    iota = lax.iota(jnp.int32, 16)

    def fire_in(src, j, blk_in, sem):
        coff = pl.multiple_of(j * 128, 128)
        pltpu.async_copy(src.at[:, pl.ds(coff, 128)], blk_in, sem)

    def wait_in(src, blk_in, sem):
        pltpu.make_async_copy(src.at[:, pl.ds(0, 128)], blk_in, sem).wait()

    def body(blk_in, blk_out, width):
        def vbody(v2l, carry):
            for half in range(2):
                col = jnp.full((16,), 2 * v2l + half, jnp.int32)
                for q in range(4):
                    vec = plsc.load_gather(blk_in, [iota + 16 * q, col])
                    blk_out[v2l, pl.ds(half * 64 + 16 * q, 16)] = vec
            return carry
        lax.fori_loop(0, width // 2, vbody, 0)

    def fire_out(dst, j, blk_out, sem):
        roff = pl.multiple_of(j * 64, 8)
        pltpu.async_copy(blk_out, dst.at[pl.ds(roff, 64)], sem)

    def drain_out(dst, blk_out, sem):
        pltpu.make_async_copy(blk_out, dst.at[pl.ds(0, 64)], sem).wait()

    def pipe(src, dst, j0, count):
        fire_in(src, j0, in_a, sem_ia)

        def pbody(p, carry):
            ja = j0 + 2 * p
            fire_in(src, ja + 1, in_b, sem_ib)
            wait_in(src, in_a, sem_ia)

            @pl.when(p > 0)
            def _():
                drain_out(dst, out_a, sem_oa)

            body(in_a, out_a, 128)
            fire_out(dst, ja, out_a, sem_oa)

            @pl.when(2 * p + 2 < count)
            def _():
                fire_in(src, ja + 2, in_a, sem_ia)

            wait_in(src, in_b, sem_ib)

            @pl.when(p > 0)
            def _():
                drain_out(dst, out_b, sem_ob)

            body(in_b, out_b, 128)
            fire_out(dst, ja + 1, out_b, sem_ob)
            return carry

        lax.fori_loop(0, count // 2, pbody, 0)
        drain_out(dst, out_a, sem_oa)
        drain_out(dst, out_b, sem_ob)

    for src, tail, dst in ((twT, tail_t, tw2), (cwT, tail_c, cw2)):
        pipe(src, dst, wid * BPT, BPT)

        @pl.when(wid < NBLK_REM)
        def _():
            j = NW * BPT + wid
            fire_in(src, j, in_a, sem_ia)
            wait_in(src, in_a, sem_ia)
            body(in_a, out_a, 128)
            fire_out(dst, j, out_a, sem_oa)
            drain_out(dst, out_a, sem_oa)

        @pl.when(wid == NW - 1)
        def _():
            pltpu.async_copy(tail, in_a, sem_ia).wait()
            body(in_a, out_a, 128)
            pltpu.async_copy(out_a.at[pl.ds(0, 32)],
                             dst.at[pl.ds(V2 - 32, 32)], sem_oa).wait()


def _sc_dots(tgt1d, ctx1d, neg1d, tw2, cw2,
             pos_hbm, negdot_hbm,
             it, ct, nt, bt, bc, bn,
             t_a, c_a, n_a, t_b, c_b, n_b,
             pos_v, neg_v, sem_a, sem_b):
    wid = lax.axis_index("s") * NC + lax.axis_index("c")
    iota = lax.iota(jnp.int32, 16)

    pltpu.sync_copy(tgt1d.at[pl.ds(wid * BPW, BPW)], it)
    pltpu.sync_copy(ctx1d.at[pl.ds(wid * BPW, BPW)], ct)
    pltpu.sync_copy(neg1d.at[pl.ds(wid * BPW * K, BPW * K)], nt)

    def prep_loop(ref, bref, n16):
        def prep(i, carry):
            v = ref[pl.ds(i * 16, 16)]
            ref[pl.ds(i * 16, 16)] = lax.shift_right_logical(v, 1)
            bref[pl.ds(i * 16, 16)] = lax.shift_left(jnp.bitwise_and(v, 1), 6)
            return carry
        lax.fori_loop(0, n16, prep, 0)

    prep_loop(it, bt, BPW // 16)
    prep_loop(ct, bc, BPW // 16)
    prep_loop(nt, bn, BPW * K // 16)

    def fire(j, t_buf, c_buf, n_buf, sem):
        pltpu.async_copy(tw2.at[it.at[pl.ds(j * C, C)]], t_buf, sem)
        pltpu.async_copy(cw2.at[ct.at[pl.ds(j * C, C)]], c_buf, sem)
        base = j * C * K
        for s, w in ((0, 128), (1, 128), (2, 64)):
            pltpu.async_copy(
                cw2.at[nt.at[pl.ds(base + s * 128, w)]],
                n_buf.at[pl.ds(s * 128, w)], sem)

    def drain(t_buf, c_buf, n_buf, sem):
        pltpu.make_async_copy(tw2.at[it.at[pl.ds(0, C)]], t_buf, sem).wait()
        pltpu.make_async_copy(cw2.at[ct.at[pl.ds(0, C)]], c_buf, sem).wait()
        for s, w in ((0, 128), (1, 128), (2, 64)):
            pltpu.make_async_copy(
                cw2.at[nt.at[pl.ds(s * 128, w)]],
                n_buf.at[pl.ds(s * 128, w)], sem).wait()

    def compute(j, t_buf, c_buf, n_buf):
        off = j * C
        zero = jnp.zeros((16,), jnp.float32)
        tcb = bt[pl.ds(off, 16)]
        ccb = bc[pl.ds(off, 16)]
        nbase = jnp.full((16,), j * C * K, jnp.int32) + iota * K

        def half_a(i, accs):
            pos, tcol, ccol = accs[0], accs[1], accs[2]
            neg = list(accs[3:3 + KH])
            ncol = list(accs[3 + KH:])
            for u in range(UNR):
                tv = plsc.load_gather(t_buf, [iota, tcol])
                cv = plsc.load_gather(c_buf, [iota, ccol])
                pos = pos + tv * cv
                for k in range(KH):
                    nv = plsc.load_gather(n_buf, [iota * K + k, ncol[k]])
                    neg[k] = neg[k] + nv * tv
                tcol = tcol + 1
                ccol = ccol + 1
                ncol = [x + 1 for x in ncol]
            return (pos, tcol, ccol, *neg, *ncol)

        def half_b(i, accs):
            tcol = accs[0]
            neg = list(accs[1:1 + KH])
            ncol = list(accs[1 + KH:])
            for u in range(UNR):
                tv = plsc.load_gather(t_buf, [iota, tcol])
                for k in range(KH):
                    nv = plsc.load_gather(n_buf, [iota * K + KH + k, ncol[k]])
                    neg[k] = neg[k] + nv * tv
                tcol = tcol + 1
                ncol = [x + 1 for x in ncol]
            return (tcol, *neg, *ncol)

        ncb_a = [plsc.load_gather(bn, [nbase + k]) for k in range(KH)]
        accs = lax.fori_loop(0, D // UNR, half_a,
                             (zero, tcb, ccb) + (zero,) * KH + tuple(ncb_a))
        pos_v[pl.ds(off, 16)] = accs[0]
        for k in range(KH):
            neg_v[k, pl.ds(off, 16)] = accs[3 + k]

        ncb_b = [plsc.load_gather(bn, [nbase + KH + k]) for k in range(KH)]
        accs_b = lax.fori_loop(0, D // UNR, half_b,
                               (tcb,) + (zero,) * KH + tuple(ncb_b))
        for k in range(KH):
            neg_v[KH + k, pl.ds(off, 16)] = accs_b[1 + k]

    fire(0, t_a, c_a, n_a, sem_a)

    def pair_body(p, carry):
        even = 2 * p
        fire(even + 1, t_b, c_b, n_b, sem_b)
        drain(t_a, c_a, n_a, sem_a)
        compute(even, t_a, c_a, n_a)

        @pl.when(p < NPAIR - 1)
        def _():
            fire(even + 2, t_a, c_a, n_a, sem_a)

        drain(t_b, c_b, n_b, sem_b)
        compute(even + 1, t_b, c_b, n_b)
        return carry

    lax.fori_loop(0, NPAIR, pair_body, 0)

    pltpu.sync_copy(pos_v, pos_hbm.at[pl.ds(wid * BPW, BPW)])
    pltpu.sync_copy(neg_v, negdot_hbm.at[wid])


def _tc_loss(pos_ref, neg_ref, out_ref):
    p = pos_ref[...]
    n = neg_ref[...]
    s1 = jnp.sum(-jax.nn.log_sigmoid(p))
    s2 = jnp.sum(-jax.nn.log_sigmoid(-n))
    out_ref[0, 0] = (s1 + s2) / B


def kernel(target, context, negative_samples, target_weight, context_weight):
    tgt1d = target.astype(jnp.int32)
    ctx1d = context.astype(jnp.int32)
    neg1d = negative_samples.astype(jnp.int32).reshape(B * K)

    tw2, cw2 = pl.kernel(
        _sc_transpose,
        out_type=(jax.ShapeDtypeStruct((V2, 128), jnp.float32),
                  jax.ShapeDtypeStruct((V2, 128), jnp.float32)),
        mesh=_mesh,
        compiler_params=_params,
        scratch_types=[
            pltpu.VMEM((64, 128), jnp.float32),
            pltpu.VMEM((64, 128), jnp.float32),
            pltpu.VMEM((64, 128), jnp.float32),
            pltpu.VMEM((64, 128), jnp.float32),
            pltpu.SemaphoreType.DMA,
            pltpu.SemaphoreType.DMA,
            pltpu.SemaphoreType.DMA,
            pltpu.SemaphoreType.DMA,
        ],
    )(target_weight.T, context_weight.T,
      jnp.pad(target_weight.T[:, V - 64:], ((0, 0), (0, 64))),
      jnp.pad(context_weight.T[:, V - 64:], ((0, 0), (0, 64))))

    pos, negdot = pl.kernel(
        _sc_dots,
        out_type=(jax.ShapeDtypeStruct((B,), jnp.float32),
                  jax.ShapeDtypeStruct((NW, K, BPW), jnp.float32)),
        mesh=_mesh,
        compiler_params=_params,
        scratch_types=[
            pltpu.VMEM((BPW,), jnp.int32),
            pltpu.VMEM((BPW,), jnp.int32),
            pltpu.VMEM((BPW * K,), jnp.int32),
            pltpu.VMEM((BPW,), jnp.int32),
            pltpu.VMEM((BPW,), jnp.int32),
            pltpu.VMEM((BPW * K,), jnp.int32),
            pltpu.VMEM((C, 128), jnp.float32),
            pltpu.VMEM((C, 128), jnp.float32),
            pltpu.VMEM((C * K, 128), jnp.float32),
            pltpu.VMEM((C, 128), jnp.float32),
            pltpu.VMEM((C, 128), jnp.float32),
            pltpu.VMEM((C * K, 128), jnp.float32),
            pltpu.VMEM((BPW,), jnp.float32),
            pltpu.VMEM((K, BPW), jnp.float32),
            pltpu.SemaphoreType.DMA,
            pltpu.SemaphoreType.DMA,
        ],
    )(tgt1d, ctx1d, neg1d, tw2, cw2)

    loss = pl.pallas_call(
        _tc_loss,
        out_shape=jax.ShapeDtypeStruct((1, 1), jnp.float32),
        out_specs=pl.BlockSpec(memory_space=pltpu.SMEM),
    )(pos.reshape(128, 128), negdot.reshape(NW * K * BPW // 128, 128))
    return loss.reshape(())


# speedup vs baseline: 5.3747x; 2.9334x over previous
import jax
import jax.numpy as jnp
from jax import lax
from jax.experimental import pallas as pl
from jax.experimental.pallas import tpu as pltpu
from jax.experimental.pallas import tpu_sc as plsc

B = 16384
K = 20
D = 64
NC = 2
NS = 16
NW = NC * NS
BPW = B // NW
C = 32
NCHUNK = BPW // C
NPAIR = NCHUNK // 2
NSEG = C * K // 128
UNR = 4


def _sc_dots(tgt1d, ctx1d, neg1d, tw, cw,
             pos_hbm, negdot_hbm,
             tgt_idx, ctx_idx, neg_idx,
             t_a, c_a, n_a, t_b, c_b, n_b,
             pos_v, neg_v, sem_a, sem_b):
    wid = lax.axis_index("s") * NC + lax.axis_index("c")
    iota = lax.iota(jnp.int32, 16)

    pltpu.sync_copy(tgt1d.at[pl.ds(wid * BPW, BPW)], tgt_idx)
    pltpu.sync_copy(ctx1d.at[pl.ds(wid * BPW, BPW)], ctx_idx)
    pltpu.sync_copy(neg1d.at[pl.ds(wid * BPW * K, BPW * K)], neg_idx)

    def fire(j, t_buf, c_buf, n_buf, sem):
        pltpu.async_copy(tw.at[tgt_idx.at[pl.ds(j * C, C)]], t_buf, sem)
        pltpu.async_copy(cw.at[ctx_idx.at[pl.ds(j * C, C)]], c_buf, sem)
        for s in range(NSEG):
            pltpu.async_copy(
                cw.at[neg_idx.at[pl.ds(j * C * K + s * 128, 128)]],
                n_buf.at[pl.ds(s * 128, 128)], sem)

    def drain(t_buf, c_buf, n_buf, sem):
        pltpu.make_async_copy(tw.at[tgt_idx.at[pl.ds(0, C)]], t_buf, sem).wait()
        pltpu.make_async_copy(cw.at[ctx_idx.at[pl.ds(0, C)]], c_buf, sem).wait()
        for s in range(NSEG):
            pltpu.make_async_copy(
                cw.at[neg_idx.at[pl.ds(s * 128, 128)]],
                n_buf.at[pl.ds(s * 128, 128)], sem).wait()

    def compute(j, t_buf, c_buf, n_buf):
        KH = K // 2
        for grp in range(C // 16):
            rowt = grp * 16 + iota
            off = j * C + grp * 16
            zero = jnp.zeros((16,), jnp.float32)
            skew0 = jnp.bitwise_and(iota * 4, 63)

            def half_a(i, accs):
                pos, dcol = accs[0], accs[1]
                neg = list(accs[2:])
                for u in range(UNR):
                    tv = plsc.load_gather(t_buf, [rowt, dcol])
                    cv = plsc.load_gather(c_buf, [rowt, dcol])
                    pos = pos + tv * cv
                    for k in range(KH):
                        nv = plsc.load_gather(n_buf, [rowt * K + k, dcol])
                        neg[k] = neg[k] + nv * tv
                    dcol = jnp.bitwise_and(dcol + 1, 63)
                return (pos, dcol, *neg)

            def half_b(i, accs):
                dcol = accs[0]
                neg = list(accs[1:])
                for u in range(UNR):
                    tv = plsc.load_gather(t_buf, [rowt, dcol])
                    for k in range(KH):
                        nv = plsc.load_gather(n_buf, [rowt * K + KH + k, dcol])
                        neg[k] = neg[k] + nv * tv
                    dcol = jnp.bitwise_and(dcol + 1, 63)
                return (dcol, *neg)

            accs = lax.fori_loop(0, D // UNR, half_a,
                                 (zero, skew0) + (zero,) * KH)
            pos_v[pl.ds(off, 16)] = accs[0]
            for k in range(KH):
                neg_v[k, pl.ds(off, 16)] = accs[k + 2]
            accs_b = lax.fori_loop(0, D // UNR, half_b, (skew0,) + (zero,) * KH)
            for k in range(KH):
                neg_v[KH + k, pl.ds(off, 16)] = accs_b[k + 1]

    fire(0, t_a, c_a, n_a, sem_a)

    def pair_body(p, carry):
        even = 2 * p
        fire(even + 1, t_b, c_b, n_b, sem_b)
        drain(t_a, c_a, n_a, sem_a)
        compute(even, t_a, c_a, n_a)

        @pl.when(p < NPAIR - 1)
        def _():
            fire(even + 2, t_a, c_a, n_a, sem_a)

        drain(t_b, c_b, n_b, sem_b)
        compute(even + 1, t_b, c_b, n_b)
        return carry

    lax.fori_loop(0, NPAIR, pair_body, 0)

    pltpu.sync_copy(pos_v, pos_hbm.at[pl.ds(wid * BPW, BPW)])
    pltpu.sync_copy(neg_v, negdot_hbm.at[wid])


def _tc_loss(pos_ref, neg_ref, out_ref):
    p = pos_ref[...]
    n = neg_ref[...]
    s1 = jnp.sum(-jax.nn.log_sigmoid(p))
    s2 = jnp.sum(-jax.nn.log_sigmoid(-n))
    out_ref[0, 0] = (s1 + s2) / B


def kernel(target, context, negative_samples, target_weight, context_weight):
    tgt1d = target.astype(jnp.int32)
    ctx1d = context.astype(jnp.int32)
    neg1d = negative_samples.astype(jnp.int32).reshape(B * K)

    mesh = plsc.VectorSubcoreMesh(
        core_axis_name="c", subcore_axis_name="s",
        num_cores=NC, num_subcores=NS)
    pos, negdot = pl.kernel(
        _sc_dots,
        out_type=(jax.ShapeDtypeStruct((B,), jnp.float32),
                  jax.ShapeDtypeStruct((NW, K, BPW), jnp.float32)),
        mesh=mesh,
        compiler_params=pltpu.CompilerParams(
            needs_layout_passes=False, use_tc_tiling_on_sc=False),
        scratch_types=[
            pltpu.VMEM((BPW,), jnp.int32),
            pltpu.VMEM((BPW,), jnp.int32),
            pltpu.VMEM((BPW * K,), jnp.int32),
            pltpu.VMEM((C, D), jnp.float32),
            pltpu.VMEM((C, D), jnp.float32),
            pltpu.VMEM((C * K, D), jnp.float32),
            pltpu.VMEM((C, D), jnp.float32),
            pltpu.VMEM((C, D), jnp.float32),
            pltpu.VMEM((C * K, D), jnp.float32),
            pltpu.VMEM((BPW,), jnp.float32),
            pltpu.VMEM((K, BPW), jnp.float32),
            pltpu.SemaphoreType.DMA,
            pltpu.SemaphoreType.DMA,
        ],
    )(tgt1d, ctx1d, neg1d, target_weight, context_weight)

    loss = pl.pallas_call(
        _tc_loss,
        out_shape=jax.ShapeDtypeStruct((1, 1), jnp.float32),
        out_specs=pl.BlockSpec(memory_space=pltpu.SMEM),
    )(pos.reshape(128, 128), negdot.reshape(NW * K * BPW // 128, 128))
    return loss.reshape(())


# speedup vs baseline: 6.2501x; 1.1629x over previous
import jax
import jax.numpy as jnp
from jax import lax
from jax.experimental import pallas as pl
from jax.experimental.pallas import tpu as pltpu
from jax.experimental.pallas import tpu_sc as plsc

V = 1000000
V2 = V // 2
B = 16384
K = 20
D = 64
NC = 2
NS = 16
NW = NC * NS
BPW = B // NW
C = 16
NCHUNK = BPW // C
NPAIR = NCHUNK // 2
UNR = 4
KH = K // 2

NBLK_FULL = V // 128
BPT = NBLK_FULL // NW
NBLK_REM = NBLK_FULL - BPT * NW

_params = pltpu.CompilerParams(
    needs_layout_passes=False, use_tc_tiling_on_sc=True)
_mesh = plsc.VectorSubcoreMesh(
    core_axis_name="c", subcore_axis_name="s", num_cores=NC, num_subcores=NS)


def _sc_transpose(twT, cwT, tail_t, tail_c, tw2, cw2,
                  in_a, in_b, out_a, out_b, sem_ia, sem_ib, sem_oa, sem_ob):
    wid = lax.axis_index("s") * NC + lax.axis_index("c")
    iota = lax.iota(jnp.int32, 16)

    def fire_in(src, j, blk_in, sem):
        coff = pl.multiple_of(j * 128, 128)
        pltpu.async_copy(src.at[:, pl.ds(coff, 128)], blk_in, sem)

    def wait_in(src, blk_in, sem):
        pltpu.make_async_copy(src.at[:, pl.ds(0, 128)], blk_in, sem).wait()

    def body(blk_in, blk_out, width):
        rowv = [8 * c + lax.shift_right_logical(iota, 1)
                for c in range(width // 16)]
        pbit = lax.shift_left(jnp.bitwise_and(iota, 1), 6)
        off0 = jnp.bitwise_and(iota * 4, 63)

        def vbody(d, off):
            col = pbit + off
            for c in range(width // 16):
                vals = blk_in[d, pl.ds(16 * c, 16)]
                plsc.store_scatter(blk_out, [rowv[c], col], vals)
            return jnp.bitwise_and(off + 1, 63)

        lax.fori_loop(0, D, vbody, off0)

    def fire_out(dst, j, blk_out, sem):
        roff = pl.multiple_of(j * 64, 8)
        pltpu.async_copy(blk_out, dst.at[pl.ds(roff, 64)], sem)

    def drain_out(dst, blk_out, sem):
        pltpu.make_async_copy(blk_out, dst.at[pl.ds(0, 64)], sem).wait()

    def pipe(src, dst, j0, count):
        fire_in(src, j0, in_a, sem_ia)

        def pbody(p, carry):
            ja = j0 + 2 * p
            fire_in(src, ja + 1, in_b, sem_ib)
            wait_in(src, in_a, sem_ia)

            @pl.when(p > 0)
            def _():
                drain_out(dst, out_a, sem_oa)

            body(in_a, out_a, 128)
            fire_out(dst, ja, out_a, sem_oa)

            @pl.when(2 * p + 2 < count)
            def _():
                fire_in(src, ja + 2, in_a, sem_ia)

            wait_in(src, in_b, sem_ib)

            @pl.when(p > 0)
            def _():
                drain_out(dst, out_b, sem_ob)

            body(in_b, out_b, 128)
            fire_out(dst, ja + 1, out_b, sem_ob)
            return carry

        lax.fori_loop(0, count // 2, pbody, 0)
        drain_out(dst, out_a, sem_oa)
        drain_out(dst, out_b, sem_ob)

    for src, tail, dst in ((twT, tail_t, tw2), (cwT, tail_c, cw2)):
        pipe(src, dst, wid * BPT, BPT)

        @pl.when(wid < NBLK_REM)
        def _():
            j = NW * BPT + wid
            fire_in(src, j, in_a, sem_ia)
            wait_in(src, in_a, sem_ia)
            body(in_a, out_a, 128)
            fire_out(dst, j, out_a, sem_oa)
            drain_out(dst, out_a, sem_oa)

        @pl.when(wid == NW - 1)
        def _():
            pltpu.async_copy(tail, in_a, sem_ia).wait()
            body(in_a, out_a, 128)
            pltpu.async_copy(out_a.at[pl.ds(0, 32)],
                             dst.at[pl.ds(V2 - 32, 32)], sem_oa).wait()


def _sc_dots(tgt1d, ctx1d, neg1d, tw2, cw2,
             pos_hbm, negdot_hbm,
             it, ct, nt, bt, bc, bn,
             t_a, c_a, n_a, t_b, c_b, n_b,
             pos_v, neg_v, sem_a, sem_b):
    wid = lax.axis_index("s") * NC + lax.axis_index("c")
    iota = lax.iota(jnp.int32, 16)

    pltpu.sync_copy(tgt1d.at[pl.ds(wid * BPW, BPW)], it)
    pltpu.sync_copy(ctx1d.at[pl.ds(wid * BPW, BPW)], ct)
    pltpu.sync_copy(neg1d.at[pl.ds(wid * BPW * K, BPW * K)], nt)

    def prep_loop(ref, bref, n16):
        def prep(i, carry):
            v = ref[pl.ds(i * 16, 16)]
            ref[pl.ds(i * 16, 16)] = lax.shift_right_logical(v, 1)
            bref[pl.ds(i * 16, 16)] = (
                lax.shift_left(jnp.bitwise_and(v, 1), 6)
                + lax.shift_left(jnp.bitwise_and(v, 15), 2))
            return carry
        lax.fori_loop(0, n16, prep, 0)

    prep_loop(it, bt, BPW // 16)
    prep_loop(ct, bc, BPW // 16)
    prep_loop(nt, bn, BPW * K // 16)

    def fire(j, t_buf, c_buf, n_buf, sem):
        pltpu.async_copy(tw2.at[it.at[pl.ds(j * C, C)]], t_buf, sem)
        pltpu.async_copy(cw2.at[ct.at[pl.ds(j * C, C)]], c_buf, sem)
        base = j * C * K
        for s, w in ((0, 128), (1, 128), (2, 64)):
            pltpu.async_copy(
                cw2.at[nt.at[pl.ds(base + s * 128, w)]],
                n_buf.at[pl.ds(s * 128, w)], sem)

    def drain(t_buf, c_buf, n_buf, sem):
        pltpu.make_async_copy(tw2.at[it.at[pl.ds(0, C)]], t_buf, sem).wait()
        pltpu.make_async_copy(cw2.at[ct.at[pl.ds(0, C)]], c_buf, sem).wait()
        for s, w in ((0, 128), (1, 128), (2, 64)):
            pltpu.make_async_copy(
                cw2.at[nt.at[pl.ds(s * 128, w)]],
                n_buf.at[pl.ds(s * 128, w)], sem).wait()

    def compute(j, t_buf, c_buf, n_buf):
        off = j * C
        zero = jnp.zeros((16,), jnp.float32)
        tcb = bt[pl.ds(off, 16)]
        ccb = bc[pl.ds(off, 16)]
        nbase = jnp.full((16,), j * C * K, jnp.int32) + iota * K
        tpb = jnp.bitwise_and(tcb, 64)
        tof0 = jnp.bitwise_and(tcb, 63)
        cpb = jnp.bitwise_and(ccb, 64)
        cof0 = jnp.bitwise_and(ccb, 63)

        def half_a(i, accs):
            pos, tof, cof = accs[0], accs[1], accs[2]
            neg = list(accs[3:3 + KH])
            nof = list(accs[3 + KH:])
            for u in range(UNR):
                tv = plsc.load_gather(t_buf, [iota, tpb + tof])
                cv = plsc.load_gather(c_buf, [iota, cpb + cof])
                pos = pos + tv * cv
                for k in range(KH):
                    nv = plsc.load_gather(
                        n_buf, [iota * K + k, npb[k] + nof[k]])
                    neg[k] = neg[k] + nv * tv
                tof = jnp.bitwise_and(tof + 1, 63)
                cof = jnp.bitwise_and(cof + 1, 63)
                nof = [jnp.bitwise_and(x + 1, 63) for x in nof]
            return (pos, tof, cof, *neg, *nof)

        def half_b(i, accs):
            tof = accs[0]
            neg = list(accs[1:1 + KH])
            nof = list(accs[1 + KH:])
            for u in range(UNR):
                tv = plsc.load_gather(t_buf, [iota, tpb + tof])
                for k in range(KH):
                    nv = plsc.load_gather(
                        n_buf, [iota * K + KH + k, npb[k] + nof[k]])
                    neg[k] = neg[k] + nv * tv
                tof = jnp.bitwise_and(tof + 1, 63)
                nof = [jnp.bitwise_and(x + 1, 63) for x in nof]
            return (tof, *neg, *nof)

        ncb = [plsc.load_gather(bn, [nbase + k]) for k in range(KH)]
        npb = [jnp.bitwise_and(x, 64) for x in ncb]
        nof0 = [jnp.bitwise_and(x, 63) for x in ncb]
        accs = lax.fori_loop(0, D // UNR, half_a,
                             (zero, tof0, cof0) + (zero,) * KH + tuple(nof0))
        pos_v[pl.ds(off, 16)] = accs[0]
        for k in range(KH):
            neg_v[k, pl.ds(off, 16)] = accs[3 + k]

        ncb = [plsc.load_gather(bn, [nbase + KH + k]) for k in range(KH)]
        npb = [jnp.bitwise_and(x, 64) for x in ncb]
        nof0 = [jnp.bitwise_and(x, 63) for x in ncb]
        accs_b = lax.fori_loop(0, D // UNR, half_b,
                               (tof0,) + (zero,) * KH + tuple(nof0))
        for k in range(KH):
            neg_v[KH + k, pl.ds(off, 16)] = accs_b[1 + k]

    fire(0, t_a, c_a, n_a, sem_a)

    def pair_body(p, carry):
        even = 2 * p
        fire(even + 1, t_b, c_b, n_b, sem_b)
        drain(t_a, c_a, n_a, sem_a)
        compute(even, t_a, c_a, n_a)

        @pl.when(p < NPAIR - 1)
        def _():
            fire(even + 2, t_a, c_a, n_a, sem_a)

        drain(t_b, c_b, n_b, sem_b)
        compute(even + 1, t_b, c_b, n_b)
        return carry

    lax.fori_loop(0, NPAIR, pair_body, 0)

    pltpu.sync_copy(pos_v, pos_hbm.at[pl.ds(wid * BPW, BPW)])
    pltpu.sync_copy(neg_v, negdot_hbm.at[wid])


def _tc_loss(pos_ref, neg_ref, out_ref):
    p = pos_ref[...]
    n = neg_ref[...]
    s1 = jnp.sum(-jax.nn.log_sigmoid(p))
    s2 = jnp.sum(-jax.nn.log_sigmoid(-n))
    out_ref[0, 0] = (s1 + s2) / B


def kernel(target, context, negative_samples, target_weight, context_weight):
    tgt1d = target.astype(jnp.int32)
    ctx1d = context.astype(jnp.int32)
    neg1d = negative_samples.astype(jnp.int32).reshape(B * K)

    tw2, cw2 = pl.kernel(
        _sc_transpose,
        out_type=(jax.ShapeDtypeStruct((V2, 128), jnp.float32),
                  jax.ShapeDtypeStruct((V2, 128), jnp.float32)),
        mesh=_mesh,
        compiler_params=_params,
        scratch_types=[
            pltpu.VMEM((64, 128), jnp.float32),
            pltpu.VMEM((64, 128), jnp.float32),
            pltpu.VMEM((64, 128), jnp.float32),
            pltpu.VMEM((64, 128), jnp.float32),
            pltpu.SemaphoreType.DMA,
            pltpu.SemaphoreType.DMA,
            pltpu.SemaphoreType.DMA,
            pltpu.SemaphoreType.DMA,
        ],
    )(target_weight.T, context_weight.T,
      jnp.pad(target_weight.T[:, V - 64:], ((0, 0), (0, 64))),
      jnp.pad(context_weight.T[:, V - 64:], ((0, 0), (0, 64))))

    pos, negdot = pl.kernel(
        _sc_dots,
        out_type=(jax.ShapeDtypeStruct((B,), jnp.float32),
                  jax.ShapeDtypeStruct((NW, K, BPW), jnp.float32)),
        mesh=_mesh,
        compiler_params=_params,
        scratch_types=[
            pltpu.VMEM((BPW,), jnp.int32),
            pltpu.VMEM((BPW,), jnp.int32),
            pltpu.VMEM((BPW * K,), jnp.int32),
            pltpu.VMEM((BPW,), jnp.int32),
            pltpu.VMEM((BPW,), jnp.int32),
            pltpu.VMEM((BPW * K,), jnp.int32),
            pltpu.VMEM((C, 128), jnp.float32),
            pltpu.VMEM((C, 128), jnp.float32),
            pltpu.VMEM((C * K, 128), jnp.float32),
            pltpu.VMEM((C, 128), jnp.float32),
            pltpu.VMEM((C, 128), jnp.float32),
            pltpu.VMEM((C * K, 128), jnp.float32),
            pltpu.VMEM((BPW,), jnp.float32),
            pltpu.VMEM((K, BPW), jnp.float32),
            pltpu.SemaphoreType.DMA,
            pltpu.SemaphoreType.DMA,
        ],
    )(tgt1d, ctx1d, neg1d, tw2, cw2)

    loss = pl.pallas_call(
        _tc_loss,
        out_shape=jax.ShapeDtypeStruct((1, 1), jnp.float32),
        out_specs=pl.BlockSpec(memory_space=pltpu.SMEM),
    )(pos.reshape(128, 128), negdot.reshape(NW * K * BPW // 128, 128))
    return loss.reshape(())
